# baseline (speedup 1.0000x reference)
import functools

import jax
import jax.numpy as jnp
from jax import lax
from jax.experimental import pallas as pl
from jax.experimental.pallas import tpu as pltpu
from jax.experimental.pallas import tpu_sc as plsc

_NC, _NS, _L = 2, 16, 16
_NW = _NC * _NS
_CH = 8


def _make_sc_add(B, S, D):
    rows_per_w = S // _NW
    n_chunks = rows_per_w // _CH
    chunk_bytes = _CH * D * 4
    mesh = plsc.VectorSubcoreMesh(core_axis_name="c", subcore_axis_name="s")

    @functools.partial(
        pl.kernel,
        out_type=jax.ShapeDtypeStruct((B, S, D), jnp.float32),
        mesh=mesh,
        scratch_types=[
            pltpu.VMEM((2, _CH, D), jnp.float32),
            pltpu.VMEM((2, B, _CH, D), jnp.float32),
            pltpu.SemaphoreType.DMA,
            pltpu.SemaphoreType.DMA,
            pltpu.SemaphoreType.DMA,
        ],
    )
    def sc_add(x_hbm, emb_hbm, out_hbm, ebuf, xbuf, esem, xisem, xosem):
        wid = lax.axis_index("s") * _NC + lax.axis_index("c")
        base = wid * rows_per_w
        last = n_chunks - 1

        def start_e(c_addr, par):
            pltpu.async_copy(
                emb_hbm.at[pl.ds(base + c_addr * _CH, _CH)], ebuf.at[par], esem
            )

        def start_xin(c_addr, b, gen):
            pltpu.async_copy(
                x_hbm.at[b, pl.ds(base + c_addr * _CH, _CH)], xbuf.at[gen, b], xisem
            )

        def start_xout(c_addr, b, gen):
            pltpu.async_copy(
                xbuf.at[gen, b], out_hbm.at[b, pl.ds(base + c_addr * _CH, _CH)], xosem
            )

        def wait_e():
            pltpu.make_async_copy(
                emb_hbm.at[pl.ds(base, _CH)], ebuf.at[0], esem
            ).wait()

        def wait_xin():
            pltpu.make_async_copy(
                x_hbm.at[0, pl.ds(base, _CH)], xbuf.at[0, 0], xisem
            ).wait()

        def wait_xout():
            pltpu.make_async_copy(
                xbuf.at[0, 0], out_hbm.at[0, pl.ds(base, _CH)], xosem
            ).wait()

        def accumulate(gen, par, bs):
            @plsc.parallel_loop(0, _CH, unroll=8)
            def _(r):
                for j in range(D // _L):
                    e = ebuf[par, r, pl.ds(j * _L, _L)]
                    for b in bs:
                        plsc.addupdate(xbuf.at[gen, b, r, pl.ds(j * _L, _L)], e)

        start_e(0, 0)
        for b in range(B):
            start_xin(0, b, 0)

        def chunk_body(c, carry):
            cn = jnp.minimum(c + 1, last)
            gen = c % 2
            gen_n = (c + 1) % 2
            wait_e()
            start_e(cn, gen_n)
            for _ in range(B):
                wait_xin()
            @pl.when(c > 0)
            def _():
                for _ in range(B):
                    wait_xout()

            for b in range(B):
                start_xin(cn, b, gen_n)
            accumulate(gen, gen, (0, 1))
            start_xout(c, 0, gen)
            start_xout(c, 1, gen)
            accumulate(gen, gen, (2, 3))
            start_xout(c, 2, gen)
            start_xout(c, 3, gen)
            return carry

        lax.fori_loop(0, n_chunks, chunk_body, 0)

        wait_e()
        for _ in range(B):
            wait_xin()
            wait_xout()

    return sc_add


def kernel(x, emb_table):
    B, S, D = x.shape
    return _make_sc_add(B, S, D)(x, emb_table[:S])


# speedup vs baseline: 1.3419x; 1.3419x over previous
import functools

import jax
import jax.numpy as jnp
from jax import lax
from jax.experimental import pallas as pl
from jax.experimental.pallas import tpu as pltpu
from jax.experimental.pallas import tpu_sc as plsc

_NC, _NS, _L = 2, 16, 16
_NW = _NC * _NS
_CH = 8


def _make_sc_add(B, S, D):
    rows_per_w = S // _NW
    n_chunks = rows_per_w // _CH
    chunk_bytes = _CH * D * 4
    mesh = plsc.VectorSubcoreMesh(core_axis_name="c", subcore_axis_name="s")

    @functools.partial(
        pl.kernel,
        out_type=jax.ShapeDtypeStruct((B, S, D), jnp.float32),
        mesh=mesh,
        scratch_types=[
            pltpu.VMEM((2, _CH, D), jnp.float32),
            pltpu.VMEM((2, B, _CH, D), jnp.float32),
            pltpu.SemaphoreType.DMA,
            pltpu.SemaphoreType.DMA,
            pltpu.SemaphoreType.DMA,
        ],
    )
    def sc_add(x_hbm, emb_hbm, out_hbm, ebuf, xbuf, esem, xisem, xosem):
        wid = lax.axis_index("s") * _NC + lax.axis_index("c")
        base = wid * rows_per_w
        last = n_chunks - 1

        def start_e(c_addr, par):
            pltpu.async_copy(
                emb_hbm.at[pl.ds(base + c_addr * _CH, _CH)], ebuf.at[par], esem
            )

        def start_xin(c_addr, b, gen):
            pltpu.async_copy(
                x_hbm.at[b, pl.ds(base + c_addr * _CH, _CH)], xbuf.at[gen, b], xisem
            )

        def start_xout(c_addr, b, gen):
            pltpu.async_copy(
                xbuf.at[gen, b], out_hbm.at[b, pl.ds(base + c_addr * _CH, _CH)], xosem
            )

        def wait_e():
            pltpu.make_async_copy(
                emb_hbm.at[pl.ds(base, _CH)], ebuf.at[0], esem
            ).wait()

        def wait_xin():
            pltpu.make_async_copy(
                x_hbm.at[0, pl.ds(base, _CH)], xbuf.at[0, 0], xisem
            ).wait()

        def wait_xout():
            pltpu.make_async_copy(
                xbuf.at[0, 0], out_hbm.at[0, pl.ds(base, _CH)], xosem
            ).wait()

        def accumulate(gen, par, bs):
            @plsc.parallel_loop(0, _CH, unroll=4)
            def _(r):
                for j in range(D // _L):
                    e = ebuf[par, r, pl.ds(j * _L, _L)]
                    for b in bs:
                        plsc.addupdate(xbuf.at[gen, b, r, pl.ds(j * _L, _L)], e)

        start_e(0, 0)
        for b in range(B):
            start_xin(0, b, 0)

        def chunk_body(c, carry):
            cn = jnp.minimum(c + 1, last)
            gen = c % 2
            gen_n = (c + 1) % 2
            wait_e()
            start_e(cn, gen_n)
            for _ in range(B):
                wait_xin()
            @pl.when(c > 0)
            def _():
                for _ in range(B):
                    wait_xout()

            for b in range(B):
                start_xin(cn, b, gen_n)
            accumulate(gen, gen, (0, 1))
            start_xout(c, 0, gen)
            start_xout(c, 1, gen)
            accumulate(gen, gen, (2, 3))
            start_xout(c, 2, gen)
            start_xout(c, 3, gen)
            return carry

        lax.fori_loop(0, n_chunks, chunk_body, 0)

        wait_e()
        for _ in range(B):
            wait_xin()
            wait_xout()

    return sc_add


def kernel(x, emb_table):
    B, S, D = x.shape
    return _make_sc_add(B, S, D)(x, emb_table[:S])
